# baseline (speedup 1.0000x reference)
import functools

import jax
import jax.numpy as jnp
from jax import lax
from jax.experimental import pallas as pl
from jax.experimental.pallas import tpu as pltpu
from jax.experimental.pallas import tpu_sc as plsc

NN = 10000
NE = 320000
DH = 128
NR = 6
NG = 64
NTILES = 32
CH = 128
NCHUNK = NE // CH
CFULL = NCHUNK // NTILES
CREM = NCHUNK - CFULL * NTILES
NNP = 10240
ROWS_PT = NNP // 16
CNT_PAD = 61440
CNT_PT = CNT_PAD // 16
NB = 25
BR = NN // NB
NROW_CH = NN // CH
NROW_REM = NN - NROW_CH * CH

_mesh = plsc.VectorSubcoreMesh(core_axis_name="c", subcore_axis_name="s")
_sc_params = pltpu.CompilerParams(needs_layout_passes=False)


def _sds(shape, dtype):
    return jax.ShapeDtypeStruct(shape, dtype)


def _sc_counts(es, z1d):
    @functools.partial(
        pl.kernel,
        compiler_params=_sc_params,
        out_type=_sds((2 * CNT_PAD,), jnp.float32),
        mesh=_mesh,
        scratch_types=[
            pltpu.VMEM((3, 3, CH), jnp.int32),
            pltpu.VMEM((3, CH), jnp.int32),
            pltpu.VMEM((CH,), jnp.float32),
            pltpu.VMEM_SHARED((CNT_PAD,), jnp.float32),
        ] + [pltpu.SemaphoreType.DMA] * 6,
    )
    def k(es_hbm, z_hbm, out_hbm, ebuf, segb, ones_v, cnt_sh,
          si0, si1, si2, sc0, sc1, sc2):
        si = (si0, si1, si2)
        sc = (sc0, sc1, sc2)
        cid = lax.axis_index("c")
        sid = lax.axis_index("s")
        w = sid * 2 + cid
        pltpu.sync_copy(z_hbm, cnt_sh.at[pl.ds(sid * CNT_PT, CNT_PT)])
        for j in range(CH // 16):
            ones_v[pl.ds(j * 16, 16)] = jnp.ones((16,), jnp.float32)
        plsc.subcore_barrier()

        def chunk_of(r):
            return w + NTILES * r

        def in_dma(r, b):
            return pltpu.make_async_copy(
                es_hbm.at[chunk_of(r)], ebuf.at[b], si[b])

        def scat_start(b):
            pltpu.async_copy(ones_v, cnt_sh.at[segb.at[b]], sc[b], add=True)

        def scat_wait(b):
            pltpu.make_async_copy(ones_v, cnt_sh.at[segb.at[b]],
                                  sc[b]).wait()

        def compute(b):
            @plsc.parallel_loop(0, CH // 16, unroll=4)
            def _(j):
                sl16 = j * 16 + lax.iota(jnp.int32, 16)
                d16 = plsc.load_gather(ebuf.at[b, 1], [sl16])
                t16 = plsc.load_gather(ebuf.at[b, 2], [sl16])
                plsc.store_scatter(segb.at[b], [sl16], d16 * NR + t16)

        in_dma(0, 0).start()
        in_dma(1, 1).start()

        def super_round(g, _):
            for u in range(3):
                r = g * 3 + u
                b = u
                b2 = (u + 2) % 3

                @pl.when(r + 2 < CFULL)
                def _():
                    in_dma(r + 2, b2).start()

                @pl.when(r - 3 >= 0)
                def _():
                    scat_wait(b)

                in_dma(r, b).wait()
                compute(b)
                scat_start(b)
            return 0

        lax.fori_loop(0, CFULL // 3, super_round, 0)
        scat_wait(0)
        scat_wait(1)
        scat_wait(2)

        @pl.when(w < CREM)
        def _():
            in_dma(CFULL, 0).start()
            in_dma(CFULL, 0).wait()
            compute(0)
            scat_start(0)
            scat_wait(0)

        plsc.subcore_barrier()
        pltpu.sync_copy(cnt_sh.at[pl.ds(sid * CNT_PT, CNT_PT)],
                        out_hbm.at[pl.ds(cid * CNT_PAD + sid * CNT_PT, CNT_PT)])

    return k(es, z1d)


def _sc_prep(es, inv):
    @functools.partial(
        pl.kernel,
        compiler_params=_sc_params,
        out_type=_sds((NCHUNK, 3, CH), jnp.int32),
        mesh=_mesh,
        scratch_types=[
            pltpu.VMEM((3, 3, CH), jnp.int32),
            pltpu.VMEM((3, 3, CH), jnp.int32),
            pltpu.VMEM((CNT_PAD,), jnp.float32),
        ] + [pltpu.SemaphoreType.DMA] * 6,
    )
    def k(es_hbm, inv_hbm, edata_out, ebuf, pk_v, inv_v,
          si0, si1, si2, so0, so1, so2):
        si = (si0, si1, si2)
        so = (so0, so1, so2)
        cid = lax.axis_index("c")
        sid = lax.axis_index("s")
        w = sid * 2 + cid
        pltpu.sync_copy(inv_hbm, inv_v)

        def chunk_of(r):
            return w + NTILES * r

        def in_dma(r, b):
            return pltpu.make_async_copy(
                es_hbm.at[chunk_of(r)], ebuf.at[b], si[b])

        def out_dma(r, b):
            return pltpu.make_async_copy(
                pk_v.at[b], edata_out.at[chunk_of(r)], so[b])

        def compute(b):
            @plsc.parallel_loop(0, CH // 16, unroll=4)
            def _(j):
                sl16 = j * 16 + lax.iota(jnp.int32, 16)
                s16v = plsc.load_gather(ebuf.at[b, 0], [sl16])
                d16 = plsc.load_gather(ebuf.at[b, 1], [sl16])
                t16 = plsc.load_gather(ebuf.at[b, 2], [sl16])
                sc16 = plsc.load_gather(inv_v, [d16 * NR + t16])
                plsc.store_scatter(pk_v.at[b, 0], [sl16], t16 * NN + s16v)
                plsc.store_scatter(pk_v.at[b, 1], [sl16], d16)
                plsc.store_scatter(pk_v.at[b, 2], [sl16],
                                   plsc.bitcast(sc16, jnp.int32))

        in_dma(0, 0).start()
        in_dma(1, 1).start()

        def super_round(g, _):
            for u in range(3):
                r = g * 3 + u
                b = u
                b2 = (u + 2) % 3

                @pl.when(r + 2 < CFULL)
                def _():
                    in_dma(r + 2, b2).start()

                @pl.when(r - 3 >= 0)
                def _():
                    out_dma(r - 3, b).wait()

                in_dma(r, b).wait()
                compute(b)
                out_dma(r, b).start()
            return 0

        lax.fori_loop(0, CFULL // 3, super_round, 0)
        out_dma(CFULL - 3, 0).wait()
        out_dma(CFULL - 2, 1).wait()
        out_dma(CFULL - 1, 2).wait()

        @pl.when(w < CREM)
        def _():
            in_dma(CFULL, 0).start()
            in_dma(CFULL, 0).wait()
            compute(0)
            out_dma(CFULL, 0).start()
            out_dma(CFULL, 0).wait()

    return k(es, inv)


def _sc_edge_pass(h2d, edata, z2d):
    @functools.partial(
        pl.kernel,
        compiler_params=_sc_params,
        out_type=_sds((2, NNP, DH), jnp.float32),
        mesh=_mesh,
        scratch_types=[
            pltpu.VMEM((3, 3, CH), jnp.int32),
            pltpu.VMEM((2, CH), jnp.int32),
            pltpu.VMEM((2, CH, DH), jnp.float32),
            pltpu.VMEM_SHARED((NNP, DH), jnp.float32),
        ] + [pltpu.SemaphoreType.DMA] * 7,
    )
    def k(h_hbm, edata_hbm, z_hbm, out_hbm, ebuf, dstb, rows_v, agg_sh,
          si0, si1, si2, sg0, sg1, ss0, ss1):
        si = (si0, si1, si2)
        sg = (sg0, sg1)
        ss = (ss0, ss1)
        cid = lax.axis_index("c")
        sid = lax.axis_index("s")
        w = sid * 2 + cid
        pltpu.sync_copy(z_hbm, agg_sh.at[pl.ds(sid * ROWS_PT, ROWS_PT)])
        plsc.subcore_barrier()

        def chunk_of(r):
            return w + NTILES * r

        def meta_dma(r, eb):
            return pltpu.make_async_copy(
                edata_hbm.at[chunk_of(r)], ebuf.at[eb], si[eb])

        def gather_dma(r, eb, rb):
            return pltpu.make_async_copy(
                h_hbm.at[ebuf.at[eb, 0]], rows_v.at[rb], sg[rb])

        def scatter_start(rb):
            pltpu.async_copy(rows_v.at[rb], agg_sh.at[dstb.at[rb]],
                             ss[rb], add=True)

        def scatter_wait(rb):
            pltpu.make_async_copy(rows_v.at[rb], agg_sh.at[dstb.at[rb]],
                                  ss[rb]).wait()

        def scale(eb, rb):
            for j in range(CH // 16):
                sl16 = j * 16 + lax.iota(jnp.int32, 16)
                dv = plsc.load_gather(ebuf.at[eb, 1], [sl16])
                plsc.store_scatter(dstb.at[rb], [sl16], dv)
            rows2 = rows_v.at[rb]
            sref = ebuf.at[eb, 2]

            @plsc.parallel_loop(0, CH, unroll=8)
            def _(i):
                ri = jnp.full((16,), i, jnp.int32)
                bc = plsc.bitcast(plsc.load_gather(sref, [ri]), jnp.float32)
                for v in range(DH // 16):
                    col = v * 16 + lax.iota(jnp.int32, 16)
                    val = plsc.load_gather(rows2, [ri, col])
                    plsc.store_scatter(rows2, [ri, col], val * bc)

        meta_dma(0, 0).start()
        meta_dma(1, 1).start()
        meta_dma(0, 0).wait()
        gather_dma(0, 0, 0).start()

        def super_round(g, _):
            for u in range(6):
                r = g * 6 + u
                eb, rb = u % 3, u % 2
                eb1, rb1 = (u + 1) % 3, (u + 1) % 2
                eb2 = (u + 2) % 3

                @pl.when(r + 1 < CFULL)
                def _():
                    meta_dma(r + 1, eb1).wait()

                @pl.when(r - 1 >= 0)
                def _():
                    scatter_wait(rb1)

                @pl.when(r + 1 < CFULL)
                def _():
                    gather_dma(r + 1, eb1, rb1).start()

                @pl.when(r + 2 < CFULL)
                def _():
                    meta_dma(r + 2, eb2).start()

                gather_dma(r, eb, rb).wait()
                scale(eb, rb)
                scatter_start(rb)
            return 0

        lax.fori_loop(0, CFULL // 6, super_round, 0)

        scatter_wait((CFULL - 1) % 2)

        @pl.when(w < CREM)
        def _():
            meta_dma(CFULL, 0).start()
            meta_dma(CFULL, 0).wait()
            gather_dma(CFULL, 0, 0).start()
            gather_dma(CFULL, 0, 0).wait()
            scale(0, 0)
            scatter_start(0)
            scatter_wait(0)

        plsc.subcore_barrier()
        pltpu.sync_copy(agg_sh.at[pl.ds(sid * ROWS_PT, ROWS_PT)],
                        out_hbm.at[cid, pl.ds(sid * ROWS_PT, ROWS_PT)])

    return k(h2d, edata, z2d)


def _sc_pool(h3, batch, z2d, z1d):
    @functools.partial(
        pl.kernel,
        compiler_params=_sc_params,
        out_type=(_sds((2, NG, DH), jnp.float32), _sds((2 * CH,), jnp.float32)),
        mesh=_mesh,
        scratch_types=[
            pltpu.VMEM((CH, DH), jnp.float32),
            pltpu.VMEM((CH,), jnp.int32),
            pltpu.VMEM((CH,), jnp.float32),
            pltpu.VMEM_SHARED((NG, DH), jnp.float32),
            pltpu.VMEM_SHARED((CH,), jnp.float32),
        ],
    )
    def k(h_hbm, batch_hbm, z2_hbm, z1_hbm, pool_out, cnt_out,
          r_v, b_v, ones_v, pool_sh, cnt_sh):
        cid = lax.axis_index("c")
        sid = lax.axis_index("s")
        w = sid * 2 + cid

        @pl.when(sid == 0)
        def _():
            pltpu.sync_copy(z2_hbm.at[pl.ds(0, NG)], pool_sh)
            pltpu.sync_copy(z1_hbm.at[pl.ds(0, CH)], cnt_sh)

        for j in range(CH // 16):
            ones_v[pl.ds(j * 16, 16)] = jnp.ones((16,), jnp.float32)
        plsc.subcore_barrier()

        def body(c):
            pltpu.sync_copy(h_hbm.at[pl.ds(c * CH, CH)], r_v)
            pltpu.sync_copy(batch_hbm.at[pl.ds(c * CH, CH)], b_v)
            pltpu.sync_copy(r_v, pool_sh.at[b_v], add=True)
            pltpu.sync_copy(ones_v, cnt_sh.at[b_v], add=True)

        nfull = NROW_CH // NTILES
        for r in range(nfull):
            body(w + NTILES * r)
        rem = NROW_CH - nfull * NTILES

        @pl.when(w < rem)
        def _():
            body(w + NTILES * nfull)

        @pl.when(w == rem)
        def _():
            pltpu.sync_copy(h_hbm.at[pl.ds(NROW_CH * CH, NROW_REM)],
                            r_v.at[pl.ds(0, NROW_REM)])
            pltpu.sync_copy(batch_hbm.at[pl.ds(NROW_CH * CH, NROW_REM)],
                            b_v.at[pl.ds(0, NROW_REM)])
            pltpu.sync_copy(r_v.at[pl.ds(0, NROW_REM)],
                            pool_sh.at[b_v.at[pl.ds(0, NROW_REM)]], add=True)
            pltpu.sync_copy(ones_v.at[pl.ds(0, NROW_REM)],
                            cnt_sh.at[b_v.at[pl.ds(0, NROW_REM)]], add=True)

        plsc.subcore_barrier()

        @pl.when(sid == 0)
        def _():
            pltpu.sync_copy(pool_sh, pool_out.at[cid])
            pltpu.sync_copy(cnt_sh, cnt_out.at[pl.ds(cid * CH, CH)])

    return k(h3, batch, z2d, z1d)


def _tc_einsum(x, W):
    def body(x_ref, w_ref, o_ref):
        o_ref[...] = jnp.dot(x_ref[...], w_ref[0],
                             preferred_element_type=jnp.float32)

    return pl.pallas_call(
        body,
        grid=(NR, NB),
        in_specs=[
            pl.BlockSpec((BR, DH), lambda r, j: (j, 0)),
            pl.BlockSpec((1, DH, DH), lambda r, j: (r, 0, 0)),
        ],
        out_specs=pl.BlockSpec((BR, DH), lambda r, j: (r * NB + j, 0)),
        out_shape=_sds((NR * NN, DH), jnp.float32),
    )(x, W)


def _tc_inv(cnt2):
    def body(c_ref, o_ref):
        o_ref[...] = 1.0 / jnp.maximum(c_ref[0] + c_ref[1], 1.0)

    return pl.pallas_call(
        body,
        out_shape=_sds((CNT_PAD // 128, 128), jnp.float32),
    )(cnt2.reshape(2, CNT_PAD // 128, 128))


def _tc_combine(p, x, root, b):
    def body(p_ref, x_ref, r_ref, b_ref, o_ref):
        acc = (p_ref[0] + p_ref[1]
               + jnp.dot(x_ref[...], r_ref[...],
                         preferred_element_type=jnp.float32)
               + b_ref[...])
        o_ref[...] = jnp.maximum(acc, 0.0)

    return pl.pallas_call(
        body,
        grid=(NB,),
        in_specs=[
            pl.BlockSpec((2, BR, DH), lambda j: (0, j, 0)),
            pl.BlockSpec((BR, DH), lambda j: (j, 0)),
            pl.BlockSpec((DH, DH), lambda j: (0, 0)),
            pl.BlockSpec((1, DH), lambda j: (0, 0)),
        ],
        out_specs=pl.BlockSpec((BR, DH), lambda j: (j, 0)),
        out_shape=_sds((NN, DH), jnp.float32),
    )(p, x, root, b.reshape(1, DH))


def _tc_head(pool2, cnt2, flags_p, wa, wb, b1, w2p, b2p):
    def body(p_ref, c_ref, f_ref, wa_ref, wb_ref, b1_ref, w2_ref, b2_ref,
             o_ref):
        cs = jnp.maximum(c_ref[0] + c_ref[1], 1.0).reshape(NG, 1)
        pooled = (p_ref[0] + p_ref[1]) / cs
        hid = (jnp.dot(pooled, wa_ref[...], preferred_element_type=jnp.float32)
               + jnp.dot(f_ref[...], wb_ref[...],
                         preferred_element_type=jnp.float32)
               + b1_ref[...])
        hid = jnp.maximum(hid, 0.0)
        o_ref[...] = (jnp.dot(hid, w2_ref[...],
                              preferred_element_type=jnp.float32)
                      + b2_ref[...])

    return pl.pallas_call(
        body,
        out_shape=_sds((NG, DH), jnp.float32),
    )(pool2, cnt2, flags_p, wa, wb, b1.reshape(1, DH), w2p, b2p)


@jax.jit
def kernel(x, edge_index, edge_type, batch, flags,
           W1, root1, b1, W2, root2, b2, W3, root3, b3,
           Wm1, bm1, Wm2, bm2):
    es = jnp.stack([edge_index[0].reshape(NCHUNK, CH),
                    edge_index[1].reshape(NCHUNK, CH),
                    edge_type.reshape(NCHUNK, CH)], axis=1)
    z2d = jnp.zeros((ROWS_PT, DH), jnp.float32)
    z1d = jnp.zeros((CNT_PT,), jnp.float32)

    cnt2 = _sc_counts(es, z1d)
    inv = _tc_inv(cnt2).reshape(CNT_PAD)
    edata = _sc_prep(es, inv)

    h = x
    for W, root, b in ((W1, root1, b1), (W2, root2, b2), (W3, root3, b3)):
        ht = _tc_einsum(h, W)
        p = _sc_edge_pass(ht, edata, z2d)
        h = _tc_combine(p, h, root, b)

    pool2, cnt_raw = _sc_pool(h, batch, z2d, z1d)
    cnt_g = cnt_raw.reshape(2, CH)[:, :NG]

    flags_p = jnp.pad(flags, ((0, 0), (0, 3)))
    wa = Wm1[:DH]
    wb = jnp.pad(Wm1[DH:], ((0, 3), (0, 0)))
    w2p = jnp.pad(Wm2, ((0, 0), (0, DH - 2)))
    b2p = jnp.pad(bm2, (0, DH - 2)).reshape(1, DH)
    out = _tc_head(pool2, cnt_g, flags_p, wa, wb, bm1, w2p, b2p)
    return out[:, :2]


# speedup vs baseline: 1.0686x; 1.0686x over previous
import functools

import jax
import jax.numpy as jnp
from jax import lax
from jax.experimental import pallas as pl
from jax.experimental.pallas import tpu as pltpu
from jax.experimental.pallas import tpu_sc as plsc

NN = 10000
NE = 320000
DH = 128
NR = 6
NG = 64
NTILES = 32
CH = 128
NCHUNK = NE // CH
CFULL = NCHUNK // NTILES
CREM = NCHUNK - CFULL * NTILES
NNP = 10240
ROWS_PT = NNP // 16
CNT_PAD = 61440
CNT_PT = CNT_PAD // 16
NB = 25
BR = NN // NB
NROW_CH = NN // CH
NROW_REM = NN - NROW_CH * CH

_mesh = plsc.VectorSubcoreMesh(core_axis_name="c", subcore_axis_name="s")
_sc_params = pltpu.CompilerParams(needs_layout_passes=False)


def _sds(shape, dtype):
    return jax.ShapeDtypeStruct(shape, dtype)


def _sc_counts(es, z1d):
    @functools.partial(
        pl.kernel,
        compiler_params=_sc_params,
        out_type=_sds((2 * CNT_PAD,), jnp.float32),
        mesh=_mesh,
        scratch_types=[
            pltpu.VMEM((3, 3, CH), jnp.int32),
            pltpu.VMEM((3, CH), jnp.int32),
            pltpu.VMEM((CH,), jnp.float32),
            pltpu.VMEM_SHARED((CNT_PAD,), jnp.float32),
        ] + [pltpu.SemaphoreType.DMA] * 6,
    )
    def k(es_hbm, z_hbm, out_hbm, ebuf, segb, ones_v, cnt_sh,
          si0, si1, si2, sc0, sc1, sc2):
        si = (si0, si1, si2)
        sc = (sc0, sc1, sc2)
        cid = lax.axis_index("c")
        sid = lax.axis_index("s")
        w = sid * 2 + cid
        pltpu.sync_copy(z_hbm, cnt_sh.at[pl.ds(sid * CNT_PT, CNT_PT)])
        for j in range(CH // 16):
            ones_v[pl.ds(j * 16, 16)] = jnp.ones((16,), jnp.float32)
        plsc.subcore_barrier()

        def chunk_of(r):
            return w + NTILES * r

        def in_dma(r, b):
            return pltpu.make_async_copy(
                es_hbm.at[chunk_of(r)], ebuf.at[b], si[b])

        def scat_start(b):
            pltpu.async_copy(ones_v, cnt_sh.at[segb.at[b]], sc[b], add=True)

        def scat_wait(b):
            pltpu.make_async_copy(ones_v, cnt_sh.at[segb.at[b]],
                                  sc[b]).wait()

        def compute(b):
            @plsc.parallel_loop(0, CH // 16, unroll=4)
            def _(j):
                sl16 = j * 16 + lax.iota(jnp.int32, 16)
                d16 = plsc.load_gather(ebuf.at[b, 1], [sl16])
                t16 = plsc.load_gather(ebuf.at[b, 2], [sl16])
                plsc.store_scatter(segb.at[b], [sl16], d16 * NR + t16)

        in_dma(0, 0).start()
        in_dma(1, 1).start()

        def super_round(g, _):
            for u in range(3):
                r = g * 3 + u
                b = u
                b2 = (u + 2) % 3

                @pl.when(r + 2 < CFULL)
                def _():
                    in_dma(r + 2, b2).start()

                @pl.when(r - 3 >= 0)
                def _():
                    scat_wait(b)

                in_dma(r, b).wait()
                compute(b)
                scat_start(b)
            return 0

        lax.fori_loop(0, CFULL // 3, super_round, 0)
        scat_wait(0)
        scat_wait(1)
        scat_wait(2)

        @pl.when(w < CREM)
        def _():
            in_dma(CFULL, 0).start()
            in_dma(CFULL, 0).wait()
            compute(0)
            scat_start(0)
            scat_wait(0)

        plsc.subcore_barrier()
        pltpu.sync_copy(cnt_sh.at[pl.ds(sid * CNT_PT, CNT_PT)],
                        out_hbm.at[pl.ds(cid * CNT_PAD + sid * CNT_PT, CNT_PT)])

    return k(es, z1d)


def _sc_prep(es, inv):
    @functools.partial(
        pl.kernel,
        compiler_params=_sc_params,
        out_type=_sds((NCHUNK, 3, CH), jnp.int32),
        mesh=_mesh,
        scratch_types=[
            pltpu.VMEM((3, 3, CH), jnp.int32),
            pltpu.VMEM((3, 3, CH), jnp.int32),
            pltpu.VMEM((CNT_PAD,), jnp.float32),
            pltpu.VMEM((CNT_PAD,), jnp.float32),
        ] + [pltpu.SemaphoreType.DMA] * 6,
    )
    def k(es_hbm, cnt_hbm, edata_out, ebuf, pk_v, c0_v, c1_v,
          si0, si1, si2, so0, so1, so2):
        si = (si0, si1, si2)
        so = (so0, so1, so2)
        cid = lax.axis_index("c")
        sid = lax.axis_index("s")
        w = sid * 2 + cid
        pltpu.sync_copy(cnt_hbm.at[pl.ds(0, CNT_PAD)], c0_v)
        pltpu.sync_copy(cnt_hbm.at[pl.ds(CNT_PAD, CNT_PAD)], c1_v)

        def chunk_of(r):
            return w + NTILES * r

        def in_dma(r, b):
            return pltpu.make_async_copy(
                es_hbm.at[chunk_of(r)], ebuf.at[b], si[b])

        def out_dma(r, b):
            return pltpu.make_async_copy(
                pk_v.at[b], edata_out.at[chunk_of(r)], so[b])

        def compute(b):
            @plsc.parallel_loop(0, CH // 16, unroll=4)
            def _(j):
                sl16 = j * 16 + lax.iota(jnp.int32, 16)
                s16v = plsc.load_gather(ebuf.at[b, 0], [sl16])
                d16 = plsc.load_gather(ebuf.at[b, 1], [sl16])
                t16 = plsc.load_gather(ebuf.at[b, 2], [sl16])
                seg16 = d16 * NR + t16
                c16 = (plsc.load_gather(c0_v, [seg16])
                       + plsc.load_gather(c1_v, [seg16]))
                sc16 = 1.0 / jnp.maximum(c16, 1.0)
                plsc.store_scatter(pk_v.at[b, 0], [sl16], t16 * NN + s16v)
                plsc.store_scatter(pk_v.at[b, 1], [sl16], d16)
                plsc.store_scatter(pk_v.at[b, 2], [sl16],
                                   plsc.bitcast(sc16, jnp.int32))

        in_dma(0, 0).start()
        in_dma(1, 1).start()

        def super_round(g, _):
            for u in range(3):
                r = g * 3 + u
                b = u
                b2 = (u + 2) % 3

                @pl.when(r + 2 < CFULL)
                def _():
                    in_dma(r + 2, b2).start()

                @pl.when(r - 3 >= 0)
                def _():
                    out_dma(r - 3, b).wait()

                in_dma(r, b).wait()
                compute(b)
                out_dma(r, b).start()
            return 0

        lax.fori_loop(0, CFULL // 3, super_round, 0)
        out_dma(CFULL - 3, 0).wait()
        out_dma(CFULL - 2, 1).wait()
        out_dma(CFULL - 1, 2).wait()

        @pl.when(w < CREM)
        def _():
            in_dma(CFULL, 0).start()
            in_dma(CFULL, 0).wait()
            compute(0)
            out_dma(CFULL, 0).start()
            out_dma(CFULL, 0).wait()

    return k(es, inv)


def _sc_edge_pass(h2d, edata, z2d):
    @functools.partial(
        pl.kernel,
        compiler_params=_sc_params,
        out_type=_sds((2, NNP, DH), jnp.float32),
        mesh=_mesh,
        scratch_types=[
            pltpu.VMEM((3, 3, CH), jnp.int32),
            pltpu.VMEM((2, CH), jnp.int32),
            pltpu.VMEM((2, CH, DH), jnp.float32),
            pltpu.VMEM_SHARED((NNP, DH), jnp.float32),
        ] + [pltpu.SemaphoreType.DMA] * 7,
    )
    def k(h_hbm, edata_hbm, z_hbm, out_hbm, ebuf, dstb, rows_v, agg_sh,
          si0, si1, si2, sg0, sg1, ss0, ss1):
        si = (si0, si1, si2)
        sg = (sg0, sg1)
        ss = (ss0, ss1)
        cid = lax.axis_index("c")
        sid = lax.axis_index("s")
        w = sid * 2 + cid
        pltpu.sync_copy(z_hbm, agg_sh.at[pl.ds(sid * ROWS_PT, ROWS_PT)])
        plsc.subcore_barrier()

        def chunk_of(r):
            return w + NTILES * r

        def meta_dma(r, eb):
            return pltpu.make_async_copy(
                edata_hbm.at[chunk_of(r)], ebuf.at[eb], si[eb])

        def gather_dma(r, eb, rb):
            return pltpu.make_async_copy(
                h_hbm.at[ebuf.at[eb, 0]], rows_v.at[rb], sg[rb])

        def scatter_start(rb):
            pltpu.async_copy(rows_v.at[rb], agg_sh.at[dstb.at[rb]],
                             ss[rb], add=True)

        def scatter_wait(rb):
            pltpu.make_async_copy(rows_v.at[rb], agg_sh.at[dstb.at[rb]],
                                  ss[rb]).wait()

        def scale(eb, rb):
            for j in range(CH // 16):
                sl16 = j * 16 + lax.iota(jnp.int32, 16)
                dv = plsc.load_gather(ebuf.at[eb, 1], [sl16])
                plsc.store_scatter(dstb.at[rb], [sl16], dv)
            rows2 = rows_v.at[rb]
            sref = ebuf.at[eb, 2]

            @plsc.parallel_loop(0, CH, unroll=4)
            def _(i):
                ri = jnp.full((16,), i, jnp.int32)
                bc = plsc.bitcast(plsc.load_gather(sref, [ri]), jnp.float32)
                for v in range(DH // 16):
                    col = v * 16 + lax.iota(jnp.int32, 16)
                    val = plsc.load_gather(rows2, [ri, col])
                    plsc.store_scatter(rows2, [ri, col], val * bc)

        meta_dma(0, 0).start()
        meta_dma(1, 1).start()
        meta_dma(0, 0).wait()
        gather_dma(0, 0, 0).start()

        def super_round(g, _):
            for u in range(6):
                r = g * 6 + u
                eb, rb = u % 3, u % 2
                eb1, rb1 = (u + 1) % 3, (u + 1) % 2
                eb2 = (u + 2) % 3

                @pl.when(r + 1 < CFULL)
                def _():
                    meta_dma(r + 1, eb1).wait()

                @pl.when(r - 1 >= 0)
                def _():
                    scatter_wait(rb1)

                @pl.when(r + 1 < CFULL)
                def _():
                    gather_dma(r + 1, eb1, rb1).start()

                @pl.when(r + 2 < CFULL)
                def _():
                    meta_dma(r + 2, eb2).start()

                gather_dma(r, eb, rb).wait()
                scale(eb, rb)
                scatter_start(rb)
            return 0

        lax.fori_loop(0, CFULL // 6, super_round, 0)

        scatter_wait((CFULL - 1) % 2)

        @pl.when(w < CREM)
        def _():
            meta_dma(CFULL, 0).start()
            meta_dma(CFULL, 0).wait()
            gather_dma(CFULL, 0, 0).start()
            gather_dma(CFULL, 0, 0).wait()
            scale(0, 0)
            scatter_start(0)
            scatter_wait(0)

        plsc.subcore_barrier()
        pltpu.sync_copy(agg_sh.at[pl.ds(sid * ROWS_PT, ROWS_PT)],
                        out_hbm.at[cid, pl.ds(sid * ROWS_PT, ROWS_PT)])

    return k(h2d, edata, z2d)


def _sc_pool(h3, batch, z2d, z1d):
    @functools.partial(
        pl.kernel,
        compiler_params=_sc_params,
        out_type=(_sds((2, NG, DH), jnp.float32), _sds((2 * CH,), jnp.float32)),
        mesh=_mesh,
        scratch_types=[
            pltpu.VMEM((CH, DH), jnp.float32),
            pltpu.VMEM((CH,), jnp.int32),
            pltpu.VMEM((CH,), jnp.float32),
            pltpu.VMEM_SHARED((NG, DH), jnp.float32),
            pltpu.VMEM_SHARED((CH,), jnp.float32),
        ],
    )
    def k(h_hbm, batch_hbm, z2_hbm, z1_hbm, pool_out, cnt_out,
          r_v, b_v, ones_v, pool_sh, cnt_sh):
        cid = lax.axis_index("c")
        sid = lax.axis_index("s")
        w = sid * 2 + cid

        @pl.when(sid == 0)
        def _():
            pltpu.sync_copy(z2_hbm.at[pl.ds(0, NG)], pool_sh)
            pltpu.sync_copy(z1_hbm.at[pl.ds(0, CH)], cnt_sh)

        for j in range(CH // 16):
            ones_v[pl.ds(j * 16, 16)] = jnp.ones((16,), jnp.float32)
        plsc.subcore_barrier()

        def body(c):
            pltpu.sync_copy(h_hbm.at[pl.ds(c * CH, CH)], r_v)
            pltpu.sync_copy(batch_hbm.at[pl.ds(c * CH, CH)], b_v)
            pltpu.sync_copy(r_v, pool_sh.at[b_v], add=True)
            pltpu.sync_copy(ones_v, cnt_sh.at[b_v], add=True)

        nfull = NROW_CH // NTILES
        for r in range(nfull):
            body(w + NTILES * r)
        rem = NROW_CH - nfull * NTILES

        @pl.when(w < rem)
        def _():
            body(w + NTILES * nfull)

        @pl.when(w == rem)
        def _():
            pltpu.sync_copy(h_hbm.at[pl.ds(NROW_CH * CH, NROW_REM)],
                            r_v.at[pl.ds(0, NROW_REM)])
            pltpu.sync_copy(batch_hbm.at[pl.ds(NROW_CH * CH, NROW_REM)],
                            b_v.at[pl.ds(0, NROW_REM)])
            pltpu.sync_copy(r_v.at[pl.ds(0, NROW_REM)],
                            pool_sh.at[b_v.at[pl.ds(0, NROW_REM)]], add=True)
            pltpu.sync_copy(ones_v.at[pl.ds(0, NROW_REM)],
                            cnt_sh.at[b_v.at[pl.ds(0, NROW_REM)]], add=True)

        plsc.subcore_barrier()

        @pl.when(sid == 0)
        def _():
            pltpu.sync_copy(pool_sh, pool_out.at[cid])
            pltpu.sync_copy(cnt_sh, cnt_out.at[pl.ds(cid * CH, CH)])

    return k(h3, batch, z2d, z1d)


def _tc_einsum(x, W):
    def body(x_ref, w_ref, o_ref):
        o_ref[...] = jnp.dot(x_ref[...], w_ref[0],
                             preferred_element_type=jnp.float32)

    return pl.pallas_call(
        body,
        grid=(NR, NB),
        in_specs=[
            pl.BlockSpec((BR, DH), lambda r, j: (j, 0)),
            pl.BlockSpec((1, DH, DH), lambda r, j: (r, 0, 0)),
        ],
        out_specs=pl.BlockSpec((BR, DH), lambda r, j: (r * NB + j, 0)),
        out_shape=_sds((NR * NN, DH), jnp.float32),
    )(x, W)


def _tc_hroot0(x, root, b):
    def body(x_ref, r_ref, b_ref, o_ref):
        o_ref[...] = (jnp.dot(x_ref[...], r_ref[...],
                              preferred_element_type=jnp.float32)
                      + b_ref[...])

    return pl.pallas_call(
        body,
        grid=(NB,),
        in_specs=[
            pl.BlockSpec((BR, DH), lambda j: (j, 0)),
            pl.BlockSpec((DH, DH), lambda j: (0, 0)),
            pl.BlockSpec((1, DH), lambda j: (0, 0)),
        ],
        out_specs=pl.BlockSpec((BR, DH), lambda j: (j, 0)),
        out_shape=_sds((NN, DH), jnp.float32),
    )(x, root, b.reshape(1, DH))


def _tc_hroot_step(p, hr_prev, root, b):
    def body(p_ref, h_ref, r_ref, b_ref, o_ref, ox_ref):
        xb = jnp.maximum(p_ref[0] + p_ref[1] + h_ref[...], 0.0)
        ox_ref[...] = xb
        o_ref[...] = (jnp.dot(xb, r_ref[...],
                              preferred_element_type=jnp.float32)
                      + b_ref[...])

    return pl.pallas_call(
        body,
        grid=(NB,),
        in_specs=[
            pl.BlockSpec((2, BR, DH), lambda j: (0, j, 0)),
            pl.BlockSpec((BR, DH), lambda j: (j, 0)),
            pl.BlockSpec((DH, DH), lambda j: (0, 0)),
            pl.BlockSpec((1, DH), lambda j: (0, 0)),
        ],
        out_specs=[
            pl.BlockSpec((BR, DH), lambda j: (j, 0)),
            pl.BlockSpec((BR, DH), lambda j: (j, 0)),
        ],
        out_shape=[_sds((NN, DH), jnp.float32), _sds((NN, DH), jnp.float32)],
    )(p, hr_prev, root, b.reshape(1, DH))


def _tc_fused_einsum(p, hr_prev, W):
    def body(p_ref, h_ref, w_ref, o_ref):
        xb = jnp.maximum(p_ref[0] + p_ref[1] + h_ref[...], 0.0)
        o_ref[...] = jnp.dot(xb, w_ref[0], preferred_element_type=jnp.float32)

    return pl.pallas_call(
        body,
        grid=(NR, NB),
        in_specs=[
            pl.BlockSpec((2, BR, DH), lambda r, j: (0, j, 0)),
            pl.BlockSpec((BR, DH), lambda r, j: (j, 0)),
            pl.BlockSpec((1, DH, DH), lambda r, j: (r, 0, 0)),
        ],
        out_specs=pl.BlockSpec((BR, DH), lambda r, j: (r * NB + j, 0)),
        out_shape=_sds((NR * NN, DH), jnp.float32),
    )(p, hr_prev, W)


def _tc_combine(p, hr):
    def body(p_ref, h_ref, o_ref):
        o_ref[...] = jnp.maximum(p_ref[0] + p_ref[1] + h_ref[...], 0.0)

    return pl.pallas_call(
        body,
        grid=(NB,),
        in_specs=[
            pl.BlockSpec((2, BR, DH), lambda j: (0, j, 0)),
            pl.BlockSpec((BR, DH), lambda j: (j, 0)),
        ],
        out_specs=pl.BlockSpec((BR, DH), lambda j: (j, 0)),
        out_shape=_sds((NN, DH), jnp.float32),
    )(p, hr)


def _tc_head(pool2, cnt2, flags_p, wa, wb, b1, w2p, b2p):
    def body(p_ref, c_ref, f_ref, wa_ref, wb_ref, b1_ref, w2_ref, b2_ref,
             o_ref):
        cs = jnp.maximum(c_ref[0] + c_ref[1], 1.0).reshape(NG, 1)
        pooled = (p_ref[0] + p_ref[1]) / cs
        hid = (jnp.dot(pooled, wa_ref[...], preferred_element_type=jnp.float32)
               + jnp.dot(f_ref[...], wb_ref[...],
                         preferred_element_type=jnp.float32)
               + b1_ref[...])
        hid = jnp.maximum(hid, 0.0)
        o_ref[...] = (jnp.dot(hid, w2_ref[...],
                              preferred_element_type=jnp.float32)
                      + b2_ref[...])

    return pl.pallas_call(
        body,
        out_shape=_sds((NG, DH), jnp.float32),
    )(pool2, cnt2, flags_p, wa, wb, b1.reshape(1, DH), w2p, b2p)


@jax.jit
def kernel(x, edge_index, edge_type, batch, flags,
           W1, root1, b1, W2, root2, b2, W3, root3, b3,
           Wm1, bm1, Wm2, bm2):
    es = jnp.stack([edge_index[0].reshape(NCHUNK, CH),
                    edge_index[1].reshape(NCHUNK, CH),
                    edge_type.reshape(NCHUNK, CH)], axis=1)
    z2d = jnp.zeros((ROWS_PT, DH), jnp.float32)
    z1d = jnp.zeros((CNT_PT,), jnp.float32)

    cnt2 = _sc_counts(es, z1d)
    edata = _sc_prep(es, cnt2)

    ht = _tc_einsum(x, W1)
    hr = _tc_hroot0(x, root1, b1)
    p = _sc_edge_pass(ht, edata, z2d)

    for W, root, b in ((W2, root2, b2), (W3, root3, b3)):
        ht = _tc_fused_einsum(p, hr, W)
        hr, _ = _tc_hroot_step(p, hr, root, b)
        p = _sc_edge_pass(ht, edata, z2d)

    h = _tc_combine(p, hr)
    pool2, cnt_raw = _sc_pool(h, batch, z2d, z1d)
    cnt_g = cnt_raw.reshape(2, CH)[:, :NG]

    flags_p = jnp.pad(flags, ((0, 0), (0, 3)))
    wa = Wm1[:DH]
    wb = jnp.pad(Wm1[DH:], ((0, 3), (0, 0)))
    w2p = jnp.pad(Wm2, ((0, 0), (0, DH - 2)))
    b2p = jnp.pad(bm2, (0, DH - 2)).reshape(1, DH)
    out = _tc_head(pool2, cnt_g, flags_p, wa, wb, bm1, w2p, b2p)
    return out[:, :2]
